# baseline (speedup 1.0000x reference)
import functools

import jax
import jax.numpy as jnp
from jax import lax
from jax.experimental import pallas as pl
from jax.experimental.pallas import tpu as pltpu
from jax.experimental.pallas import tpu_sc as plsc


@functools.lru_cache(maxsize=None)
def _make_gather(N, D, NC, NS, C):
    NW = NC * NS
    n_per_w = N // NW
    n_chunks = n_per_w // C
    mesh = plsc.VectorSubcoreMesh(core_axis_name="c", subcore_axis_name="s")

    @functools.partial(
        pl.kernel,
        mesh=mesh,
        compiler_params=pltpu.CompilerParams(use_tc_tiling_on_sc=False),
        out_type=jax.ShapeDtypeStruct((N, D), jnp.float32),
        scratch_types=[
            pltpu.VMEM((n_chunks, C), jnp.int32),
            pltpu.VMEM((C, D), jnp.float32),
            pltpu.SemaphoreType.DMA,
        ],
    )
    def k(idx_hbm, table_hbm, out_hbm, idx_v, rows_v, gsem):
        wid = lax.axis_index("s") * NC + lax.axis_index("c")
        base = wid * n_per_w
        pltpu.sync_copy(idx_hbm.at[wid], idx_v)

        def body(j, carry):
            pltpu.async_copy(table_hbm.at[idx_v.at[j]], rows_v, gsem).wait()
            pltpu.sync_copy(rows_v, out_hbm.at[pl.ds(base + j * C, C)])
            return carry

        lax.fori_loop(0, n_chunks, body, 0)

    return k


def kernel(input, table):
    B, H = input.shape
    V, D = table.shape
    N = B * H
    info = plsc.get_sparse_core_info()
    NC, NS = info.num_cores, info.num_subcores
    C = 128
    idx = input.reshape(NC * NS, N // (NC * NS) // C, C).astype(jnp.int32)
    out = _make_gather(N, D, NC, NS, C)(idx, table)
    return out.reshape(B, H, D)


# speedup vs baseline: 1.0402x; 1.0402x over previous
import functools

import jax
import jax.numpy as jnp
from jax import lax
from jax.experimental import pallas as pl
from jax.experimental.pallas import tpu as pltpu
from jax.experimental.pallas import tpu_sc as plsc


@functools.lru_cache(maxsize=None)
def _make_gather(N, D, NC, NS, C, NBUF):
    NW = NC * NS
    n_per_w = N // NW
    n_chunks = n_per_w // C
    assert n_chunks % NBUF == 0
    rounds = n_chunks // NBUF
    mesh = plsc.VectorSubcoreMesh(core_axis_name="c", subcore_axis_name="s")

    @functools.partial(
        pl.kernel,
        mesh=mesh,
        compiler_params=pltpu.CompilerParams(use_tc_tiling_on_sc=False),
        out_type=jax.ShapeDtypeStruct((N, D), jnp.float32),
        scratch_types=[
            pltpu.VMEM((n_chunks, C), jnp.int32),
            *[pltpu.VMEM((C, D), jnp.float32) for _ in range(NBUF)],
            *[pltpu.SemaphoreType.DMA for _ in range(2 * NBUF)],
        ],
    )
    def k(idx_hbm, table_hbm, out_hbm, idx_v, *rest):
        bufs = rest[:NBUF]
        gsems = rest[NBUF:2 * NBUF]
        wsems = rest[2 * NBUF:]
        wid = lax.axis_index("s") * NC + lax.axis_index("c")
        base = wid * n_per_w
        pltpu.sync_copy(idx_hbm.at[wid], idx_v)

        def fire_gather(j, b):
            pltpu.async_copy(table_hbm.at[idx_v.at[j]], bufs[b], gsems[b])

        def fire_write(j, b):
            pltpu.async_copy(bufs[b], out_hbm.at[pl.ds(base + j * C, C)],
                             wsems[b])

        for b in range(NBUF):
            fire_gather(b, b)

        def body(g, carry):
            for b in range(NBUF):
                j = g * NBUF + b
                pltpu.make_async_copy(
                    table_hbm.at[idx_v.at[j]], bufs[b], gsems[b]).wait()
                fire_write(j, b)
            for b in range(NBUF):
                j = g * NBUF + b
                pltpu.make_async_copy(
                    bufs[b], out_hbm.at[pl.ds(base + j * C, C)],
                    wsems[b]).wait()

                @pl.when(g + 1 < rounds)
                def _(b=b):
                    fire_gather((g + 1) * NBUF + b, b)
            return carry

        lax.fori_loop(0, rounds, body, 0)

    return k


def kernel(input, table):
    B, H = input.shape
    V, D = table.shape
    N = B * H
    info = plsc.get_sparse_core_info()
    NC, NS = info.num_cores, info.num_subcores
    C = 128
    NBUF = 5
    idx = input.reshape(NC * NS, N // (NC * NS) // C, C).astype(jnp.int32)
    out = _make_gather(N, D, NC, NS, C, NBUF)(idx, table)
    return out.reshape(B, H, D)


# speedup vs baseline: 1.0450x; 1.0046x over previous
import functools

import jax
import jax.numpy as jnp
from jax import lax
from jax.experimental import pallas as pl
from jax.experimental.pallas import tpu as pltpu
from jax.experimental.pallas import tpu_sc as plsc


@functools.lru_cache(maxsize=None)
def _make_gather(N, D, NC, NS, C, NBUF):
    NW = NC * NS
    n_per_w = N // NW
    n_chunks = n_per_w // C
    assert n_chunks % NBUF == 0
    rounds = n_chunks // NBUF
    mesh = plsc.VectorSubcoreMesh(core_axis_name="c", subcore_axis_name="s")

    @functools.partial(
        pl.kernel,
        mesh=mesh,
        compiler_params=pltpu.CompilerParams(use_tc_tiling_on_sc=False),
        out_type=jax.ShapeDtypeStruct((N, D), jnp.float32),
        scratch_types=[
            pltpu.VMEM((n_chunks, C), jnp.int32),
            *[pltpu.VMEM((C, D), jnp.float32) for _ in range(NBUF)],
            *[pltpu.SemaphoreType.DMA for _ in range(2 * NBUF)],
        ],
    )
    def k(idx_hbm, table_hbm, out_hbm, idx_v, *rest):
        bufs = rest[:NBUF]
        gsems = rest[NBUF:2 * NBUF]
        wsems = rest[2 * NBUF:]
        wid = lax.axis_index("s") * NC + lax.axis_index("c")
        base = wid * n_per_w
        pltpu.sync_copy(idx_hbm.at[wid], idx_v)

        def fire_gather(j, b):
            pltpu.async_copy(table_hbm.at[idx_v.at[j]], bufs[b], gsems[b])

        def fire_write(j, b):
            pltpu.async_copy(bufs[b], out_hbm.at[pl.ds(base + j * C, C)],
                             wsems[b])

        for b in range(NBUF):
            fire_gather(b, b)

        def body(g, carry):
            for b in range(NBUF):
                j = g * NBUF + b
                pltpu.make_async_copy(
                    table_hbm.at[idx_v.at[j]], bufs[b], gsems[b]).wait()
                fire_write(j, b)
            for b in range(NBUF):
                j = g * NBUF + b
                pltpu.make_async_copy(
                    bufs[b], out_hbm.at[pl.ds(base + j * C, C)],
                    wsems[b]).wait()

                @pl.when(g + 1 < rounds)
                def _(b=b):
                    fire_gather((g + 1) * NBUF + b, b)
            return carry

        lax.fori_loop(0, rounds, body, 0)

    return k


def kernel(input, table):
    B, H = input.shape
    V, D = table.shape
    N = B * H
    info = plsc.get_sparse_core_info()
    NC, NS = info.num_cores, info.num_subcores
    C = 128
    NBUF = 5
    zero_i = input[0, 0] * 0
    idx = input.astype(jnp.int32).reshape(
        NC * NS, N // (NC * NS) // C, C) + zero_i
    out = _make_gather(N, D, NC, NS, C, NBUF)(idx, table)
    return out.reshape(B, H, D) + zero_i.astype(jnp.float32)
